# baseline (speedup 1.0000x reference)
import functools

import jax
import jax.numpy as jnp
from jax import lax
from jax.experimental import pallas as pl
from jax.experimental.pallas import tpu as pltpu
from jax.experimental.pallas import tpu_sc as plsc

N = 10000
E = 320000
L = 128

_NC = 2
_NS = 16
_NW = _NC * _NS

_CHUNK = 128
_NBUF = 3
_NCHUNK = E // _CHUNK
_PER_W = -(-_NCHUNK // _NW)
_REM = _NCHUNK - (_PER_W - 1) * _NW

_NPAD = 10240
_ROWS_PER_SUB = _NPAD // _NS

_EBLK = 2560
_NHALF = 2
_EH = E // _NHALF
_EBLK2 = 2000


def _mesh():
    return plsc.VectorSubcoreMesh(core_axis_name="c", subcore_axis_name="s")


def _proj_body(n_ref, ws_ref, wr_ref, b_ref, s_out, r_out):
    x = n_ref[...]
    s_out[...] = jnp.dot(x, ws_ref[...], preferred_element_type=jnp.float32)
    r_out[...] = (
        jnp.dot(x, wr_ref[...], preferred_element_type=jnp.float32) + b_ref[...]
    )


def _proj(node, ws, wr, b):
    return pl.pallas_call(
        _proj_body,
        out_shape=(
            jax.ShapeDtypeStruct((N, L), jnp.float32),
            jax.ShapeDtypeStruct((N, L), jnp.float32),
        ),
    )(node, ws, wr, b)


def _make_gather_body(n_edges):
  nchunk = n_edges // _CHUNK
  per_w = -(-nchunk // _NW)
  rem = nchunk - (per_w - 1) * _NW

  def _gather_body(sproj, rproj, senders, receivers, out,
                   sidx, ridx, sbuf, rbuf, semi, semg, semw):
    wid = lax.axis_index("s") * _NC + lax.axis_index("c")
    nw = jnp.where(wid < rem, per_w, per_w - 1)

    def idx_load(j, slot):
        base = (wid + _NW * j) * _CHUNK
        pltpu.async_copy(senders.at[pl.ds(base, _CHUNK)], sidx[slot], semi[slot])
        pltpu.async_copy(receivers.at[pl.ds(base, _CHUNK)], ridx[slot], semi[slot])

    def gathers(j, slot):
        base = (wid + _NW * j) * _CHUNK
        pltpu.make_async_copy(
            senders.at[pl.ds(base, _CHUNK)], sidx[slot], semi[slot]).wait()
        pltpu.make_async_copy(
            receivers.at[pl.ds(base, _CHUNK)], ridx[slot], semi[slot]).wait()
        pltpu.async_copy(sproj.at[sidx[slot]], sbuf[slot], semg[slot])
        pltpu.async_copy(rproj.at[ridx[slot]], rbuf[slot], semg[slot])

    def process(j, slot):
        base = (wid + _NW * j) * _CHUNK
        pltpu.make_async_copy(sproj.at[sidx[slot]], sbuf[slot], semg[slot]).wait()
        pltpu.make_async_copy(rproj.at[ridx[slot]], rbuf[slot], semg[slot]).wait()

        def row_body(r, carry2):
            for q in range(L // 16):
                sl = pl.ds(q * 16, 16)
                plsc.addupdate(sbuf[slot].at[r, sl], rbuf[slot][r, sl])
            return carry2

        lax.fori_loop(0, _CHUNK, row_body, 0)
        pltpu.async_copy(sbuf[slot], out.at[pl.ds(base, _CHUNK)], semw[slot])

    idx_load(0, 0)
    idx_load(1, 1)
    idx_load(2, 2)
    gathers(0, 0)
    gathers(1, 1)

    def triple_body(p, carry):
        for k in range(_NBUF):
            j = _NBUF * p + k
            jn = j + 2
            slot_n = (k + 2) % _NBUF

            @pl.when(jn < nw)
            def _():
                @pl.when(jn >= _NBUF)
                def _():
                    pltpu.make_async_copy(
                        sbuf[slot_n], out.at[pl.ds(0, _CHUNK)], semw[slot_n]
                    ).wait()

                gathers(jn, slot_n)

            @pl.when(j < nw)
            def _():
                process(j, k)

            @pl.when(j + _NBUF < nw)
            def _():
                idx_load(j + _NBUF, k)

        return carry

    lax.fori_loop(0, -(-per_w // _NBUF), triple_body, 0)

    for sl in range(_NBUF):
        pltpu.make_async_copy(
            sbuf[sl], out.at[pl.ds(0, _CHUNK)], semw[sl]
        ).wait()

  return _gather_body


def _gather_sum(sproj, rproj, senders, receivers, n_edges):
    k = functools.partial(
        pl.kernel,
        out_type=jax.ShapeDtypeStruct((n_edges, L), jnp.float32),
        mesh=_mesh(),
        scratch_types=[
            [pltpu.VMEM((_CHUNK,), jnp.int32)] * _NBUF,
            [pltpu.VMEM((_CHUNK,), jnp.int32)] * _NBUF,
            [pltpu.VMEM((_CHUNK, L), jnp.float32)] * _NBUF,
            [pltpu.VMEM((_CHUNK, L), jnp.float32)] * _NBUF,
            [pltpu.SemaphoreType.DMA] * _NBUF,
            [pltpu.SemaphoreType.DMA] * _NBUF,
            [pltpu.SemaphoreType.DMA] * _NBUF,
        ],
    )(_make_gather_body(n_edges))
    return k(sproj, rproj, senders, receivers)


def _edge_body(sr_ref, e_ref, wee_ref, we2_ref, b2_ref, g_ref, bet_ref,
               ne_ref, oe_ref):
    e = e_ref[...]
    h = sr_ref[...] + jnp.dot(e, wee_ref[...], preferred_element_type=jnp.float32)
    h = jnp.maximum(h, 0.0)
    h2 = jnp.dot(h, we2_ref[...], preferred_element_type=jnp.float32) + b2_ref[...]
    h2 = jnp.maximum(h2, 0.0)
    m = jnp.mean(h2, axis=-1, keepdims=True)
    cdev = h2 - m
    v = jnp.mean(cdev * cdev, axis=-1, keepdims=True)
    ne = cdev * lax.rsqrt(v + 1e-5) * g_ref[...] + bet_ref[...]
    ne_ref[...] = ne
    oe_ref[...] = ne + e


def _edge_mlp_half(sr, edge_h, wee, we2, b2, g, beta, half, oe_prev=None):
    nblk = _EH // _EBLK2
    grid = (nblk,)
    blk = lambda i: (i, 0)
    zero = lambda i: (0, 0)
    oe_map = lambda i: (i + half * nblk, 0)
    in_specs = [
        pl.BlockSpec((_EBLK2, L), blk),
        pl.BlockSpec((_EBLK2, L), blk),
        pl.BlockSpec((L, L), zero),
        pl.BlockSpec((L, L), zero),
        pl.BlockSpec((1, L), zero),
        pl.BlockSpec((1, L), zero),
        pl.BlockSpec((1, L), zero),
    ]
    args = [sr, edge_h, wee, we2, b2, g, beta]
    kwargs = {}
    if oe_prev is not None:
        in_specs.append(pl.BlockSpec(memory_space=pl.ANY))
        args.append(oe_prev)
        kwargs["input_output_aliases"] = {7: 1}

    def body(sr_ref, e_ref, wee_ref, we2_ref, b2_ref, g_ref, bet_ref,
             *rest):
        ne_ref, oe_ref = rest[-2], rest[-1]
        _edge_body(sr_ref, e_ref, wee_ref, we2_ref, b2_ref, g_ref, bet_ref,
                   ne_ref, oe_ref)

    return pl.pallas_call(
        body,
        grid=grid,
        in_specs=in_specs,
        out_specs=[
            pl.BlockSpec((_EBLK2, L), blk),
            pl.BlockSpec((_EBLK2, L), oe_map),
        ],
        out_shape=(
            jax.ShapeDtypeStruct((_EH, L), jnp.float32),
            jax.ShapeDtypeStruct((E, L), jnp.float32),
        ),
        **kwargs,
    )(*args)


def _make_scatter_body(n_edges):
  nchunk = n_edges // _CHUNK
  per_w = -(-nchunk // _NW)
  rem = nchunk - (per_w - 1) * _NW

  def _scatter_body(ne, receivers, p0, p1, ridx, buf, acc, semi, sem):
    cid = lax.axis_index("c")
    sid = lax.axis_index("s")
    wid = sid * _NC + cid

    def zrow(r, carry):
        for q in range(L // 16):
            buf[0][r, pl.ds(q * 16, 16)] = jnp.zeros((16,), jnp.float32)
        return carry

    lax.fori_loop(0, _CHUNK, zrow, 0)
    for kk in range(_ROWS_PER_SUB // _CHUNK):
        pltpu.sync_copy(
            buf[0], acc.at[pl.ds(sid * _ROWS_PER_SUB + kk * _CHUNK, _CHUNK)]
        )
    plsc.subcore_barrier()

    nw = jnp.where(wid < rem, per_w, per_w - 1)

    def issue(j, slot):
        base = (wid + _NW * j) * _CHUNK
        pltpu.async_copy(receivers.at[pl.ds(base, _CHUNK)], ridx[slot], semi[slot])
        pltpu.async_copy(ne.at[pl.ds(base, _CHUNK)], buf[slot], sem[slot])

    def process(j, slot):
        base = (wid + _NW * j) * _CHUNK
        pltpu.make_async_copy(
            receivers.at[pl.ds(base, _CHUNK)], ridx[slot], semi[slot]
        ).wait()
        pltpu.make_async_copy(
            ne.at[pl.ds(base, _CHUNK)], buf[slot], sem[slot]
        ).wait()
        pltpu.sync_copy(buf[slot], acc.at[ridx[slot]], add=True)

    issue(0, 0)

    def pair_body(p, carry):
        for k in range(2):
            j = 2 * p + k
            jn = j + 1

            @pl.when(jn < nw)
            def _():
                issue(jn, 1 - k)

            @pl.when(j < nw)
            def _():
                process(j, k)

        return carry

    lax.fori_loop(0, -(-per_w // 2), pair_body, 0)
    plsc.subcore_barrier()

    for kk in range(_ROWS_PER_SUB // _CHUNK):
        r0 = sid * _ROWS_PER_SUB + kk * _CHUNK

        @pl.when(cid == 0)
        def _():
            pltpu.sync_copy(acc.at[pl.ds(r0, _CHUNK)], p0.at[pl.ds(r0, _CHUNK)])

        @pl.when(cid == 1)
        def _():
            pltpu.sync_copy(acc.at[pl.ds(r0, _CHUNK)], p1.at[pl.ds(r0, _CHUNK)])

  return _scatter_body


def _scatter_add(ne, receivers, n_edges):
    k = functools.partial(
        pl.kernel,
        out_type=(
            jax.ShapeDtypeStruct((_NPAD, L), jnp.float32),
            jax.ShapeDtypeStruct((_NPAD, L), jnp.float32),
        ),
        mesh=_mesh(),
        scratch_types=[
            [pltpu.VMEM((_CHUNK,), jnp.int32)] * 2,
            [pltpu.VMEM((_CHUNK, L), jnp.float32)] * 2,
            pltpu.VMEM_SHARED((_NPAD, L), jnp.float32),
            [pltpu.SemaphoreType.DMA] * 2,
            [pltpu.SemaphoreType.DMA] * 2,
        ],
    )(_make_scatter_body(n_edges))
    return k(ne, receivers)


def _node_body(n_ref, p0_ref, p1_ref, p2_ref, p3_ref, wna_ref, wnb_ref,
               b1_ref, w2_ref, b2_ref, g_ref, bet_ref, out_ref):
    x = n_ref[...]
    aggr = (p0_ref[0:N, :] + p1_ref[0:N, :]) + (p2_ref[0:N, :] + p3_ref[0:N, :])
    h = (
        jnp.dot(x, wna_ref[...], preferred_element_type=jnp.float32)
        + jnp.dot(aggr, wnb_ref[...], preferred_element_type=jnp.float32)
        + b1_ref[...]
    )
    h = jnp.maximum(h, 0.0)
    h2 = jnp.dot(h, w2_ref[...], preferred_element_type=jnp.float32) + b2_ref[...]
    h2 = jnp.maximum(h2, 0.0)
    m = jnp.mean(h2, axis=-1, keepdims=True)
    cdev = h2 - m
    v = jnp.mean(cdev * cdev, axis=-1, keepdims=True)
    out_ref[...] = cdev * lax.rsqrt(v + 1e-5) * g_ref[...] + bet_ref[...] + x


def _node_mlp(node, partials, wna, wnb, b1, w2, b2, g, beta):
    return pl.pallas_call(
        _node_body,
        out_shape=jax.ShapeDtypeStruct((N, L), jnp.float32),
    )(node, *partials, wna, wnb, b1, w2, b2, g, beta)


def kernel(node_latents, mesh_edge_latents, senders, receivers,
           W_e1, b_e1, W_e2, b_e2, g_e, beta_e,
           W_n1, b_n1, W_n2, b_n2, g_n, beta_n):
    node = node_latents.reshape(N, L)
    edge = mesh_edge_latents.reshape(E, L)
    snd = senders.astype(jnp.int32)
    rcv = receivers.astype(jnp.int32)

    ws = W_e1[0:L, :]
    wr = W_e1[L:2 * L, :]
    wee = W_e1[2 * L:3 * L, :]

    b2e = b_e2.reshape(1, L)
    ge = g_e.reshape(1, L)
    be = beta_e.reshape(1, L)

    sproj, rproj = _proj(node, ws, wr, b_e1.reshape(1, L))
    sr_a = _gather_sum(sproj, rproj, snd[0:_EH], rcv[0:_EH], _EH)
    sr_b = _gather_sum(sproj, rproj, snd[_EH:E], rcv[_EH:E], _EH)
    ne_a, oe_a = _edge_mlp_half(sr_a, edge[0:_EH], wee, W_e2, b2e, ge, be, 0)
    p0a, p1a = _scatter_add(ne_a, rcv[0:_EH], _EH)
    ne_b, new_edge = _edge_mlp_half(sr_b, edge[_EH:E], wee, W_e2, b2e, ge,
                                    be, 1, oe_prev=oe_a)
    p0b, p1b = _scatter_add(ne_b, rcv[_EH:E], _EH)
    new_node = _node_mlp(node, (p0a, p1a, p0b, p1b), W_n1[0:L, :],
                         W_n1[L:2 * L, :], b_n1.reshape(1, L), W_n2,
                         b_n2.reshape(1, L), g_n.reshape(1, L),
                         beta_n.reshape(1, L))
    return new_node.reshape(1, N, L), new_edge.reshape(1, E, L)


# speedup vs baseline: 1.1517x; 1.1517x over previous
import functools

import jax
import jax.numpy as jnp
from jax import lax
from jax.experimental import pallas as pl
from jax.experimental.pallas import tpu as pltpu
from jax.experimental.pallas import tpu_sc as plsc

N = 10000
E = 320000
L = 128

_NC = 2
_NS = 16
_NW = _NC * _NS

_CHUNK = 128
_NBUF = 3
_NCHUNK = E // _CHUNK
_PER_W = -(-_NCHUNK // _NW)
_REM = _NCHUNK - (_PER_W - 1) * _NW

_NPAD = 10240
_ROWS_PER_SUB = _NPAD // _NS

_EBLK = 2560
_NHALF = 2
_EH = E // _NHALF
_EBLK2 = 2000


def _mesh():
    return plsc.VectorSubcoreMesh(core_axis_name="c", subcore_axis_name="s")


def _proj_body(n_ref, ws_ref, wr_ref, b_ref, s_out, r_out):
    x = n_ref[...]
    s_out[...] = jnp.dot(x, ws_ref[...], preferred_element_type=jnp.float32)
    r_out[...] = (
        jnp.dot(x, wr_ref[...], preferred_element_type=jnp.float32) + b_ref[...]
    )


def _proj(node, ws, wr, b):
    return pl.pallas_call(
        _proj_body,
        out_shape=(
            jax.ShapeDtypeStruct((N, L), jnp.float32),
            jax.ShapeDtypeStruct((N, L), jnp.float32),
        ),
    )(node, ws, wr, b)


def _make_gather_body(n_edges):
  nchunk = n_edges // _CHUNK
  per_w = -(-nchunk // _NW)
  rem = nchunk - (per_w - 1) * _NW

  def _gather_body(sproj, rproj, senders, receivers, out,
                   sidx, ridx, sbuf, rbuf, semi, semg, semw):
    wid = lax.axis_index("s") * _NC + lax.axis_index("c")
    nw = jnp.where(wid < rem, per_w, per_w - 1)

    def idx_load(j, slot):
        base = (wid + _NW * j) * _CHUNK
        pltpu.async_copy(senders.at[pl.ds(base, _CHUNK)], sidx[slot], semi[slot])
        pltpu.async_copy(receivers.at[pl.ds(base, _CHUNK)], ridx[slot], semi[slot])

    def gathers(j, slot):
        base = (wid + _NW * j) * _CHUNK
        pltpu.make_async_copy(
            senders.at[pl.ds(base, _CHUNK)], sidx[slot], semi[slot]).wait()
        pltpu.make_async_copy(
            receivers.at[pl.ds(base, _CHUNK)], ridx[slot], semi[slot]).wait()
        pltpu.async_copy(sproj.at[sidx[slot]], sbuf[slot], semg[slot])
        pltpu.async_copy(rproj.at[ridx[slot]], rbuf[slot], semg[slot])

    def process(j, slot):
        base = (wid + _NW * j) * _CHUNK
        pltpu.make_async_copy(sproj.at[sidx[slot]], sbuf[slot], semg[slot]).wait()
        pltpu.make_async_copy(rproj.at[ridx[slot]], rbuf[slot], semg[slot]).wait()

        def row_body(r, carry2):
            for q in range(L // 16):
                sl = pl.ds(q * 16, 16)
                plsc.addupdate(sbuf[slot].at[r, sl], rbuf[slot][r, sl])
            return carry2

        lax.fori_loop(0, _CHUNK, row_body, 0)
        pltpu.async_copy(sbuf[slot], out.at[pl.ds(base, _CHUNK)], semw[slot])

    idx_load(0, 0)
    idx_load(1, 1)
    idx_load(2, 2)
    gathers(0, 0)
    gathers(1, 1)

    def triple_body(p, carry):
        for k in range(_NBUF):
            j = _NBUF * p + k
            jn = j + 2
            slot_n = (k + 2) % _NBUF

            @pl.when(jn < nw)
            def _():
                @pl.when(jn >= _NBUF)
                def _():
                    pltpu.make_async_copy(
                        sbuf[slot_n], out.at[pl.ds(0, _CHUNK)], semw[slot_n]
                    ).wait()

                gathers(jn, slot_n)

            @pl.when(j < nw)
            def _():
                process(j, k)

            @pl.when(j + _NBUF < nw)
            def _():
                idx_load(j + _NBUF, k)

        return carry

    lax.fori_loop(0, -(-per_w // _NBUF), triple_body, 0)

    for sl in range(_NBUF):
        pltpu.make_async_copy(
            sbuf[sl], out.at[pl.ds(0, _CHUNK)], semw[sl]
        ).wait()

  return _gather_body


def _gather_sum(sproj, rproj, senders, receivers, n_edges):
    k = functools.partial(
        pl.kernel,
        out_type=jax.ShapeDtypeStruct((n_edges, L), jnp.float32),
        mesh=_mesh(),
        scratch_types=[
            [pltpu.VMEM((_CHUNK,), jnp.int32)] * _NBUF,
            [pltpu.VMEM((_CHUNK,), jnp.int32)] * _NBUF,
            [pltpu.VMEM((_CHUNK, L), jnp.float32)] * _NBUF,
            [pltpu.VMEM((_CHUNK, L), jnp.float32)] * _NBUF,
            [pltpu.SemaphoreType.DMA] * _NBUF,
            [pltpu.SemaphoreType.DMA] * _NBUF,
            [pltpu.SemaphoreType.DMA] * _NBUF,
        ],
    )(_make_gather_body(n_edges))
    return k(sproj, rproj, senders, receivers)


def _edge_body(sr_ref, e_ref, wee_ref, we2_ref, b2_ref, g_ref, bet_ref,
               ne_ref, oe_ref):
    e = e_ref[...]
    h = sr_ref[...] + jnp.dot(
        e.astype(jnp.bfloat16), wee_ref[...].astype(jnp.bfloat16),
        preferred_element_type=jnp.float32)
    h = jnp.maximum(h, 0.0)
    h2 = jnp.dot(
        h.astype(jnp.bfloat16), we2_ref[...].astype(jnp.bfloat16),
        preferred_element_type=jnp.float32) + b2_ref[...]
    h2 = jnp.maximum(h2, 0.0)
    m = jnp.mean(h2, axis=-1, keepdims=True)
    cdev = h2 - m
    v = jnp.mean(cdev * cdev, axis=-1, keepdims=True)
    ne = cdev * lax.rsqrt(v + 1e-5) * g_ref[...] + bet_ref[...]
    ne_ref[...] = ne
    oe_ref[...] = ne + e


def _edge_mlp(sr, edge, wee, we2, b2, g, beta):
    grid = (E // _EBLK,)
    blk = lambda i: (i, 0)
    zero = lambda i: (0, 0)
    return pl.pallas_call(
        _edge_body,
        grid=grid,
        in_specs=[
            pl.BlockSpec((_EBLK, L), blk),
            pl.BlockSpec((_EBLK, L), blk),
            pl.BlockSpec((L, L), zero),
            pl.BlockSpec((L, L), zero),
            pl.BlockSpec((1, L), zero),
            pl.BlockSpec((1, L), zero),
            pl.BlockSpec((1, L), zero),
        ],
        out_specs=[
            pl.BlockSpec((_EBLK, L), blk),
            pl.BlockSpec((_EBLK, L), blk),
        ],
        out_shape=(
            jax.ShapeDtypeStruct((E, L), jnp.float32),
            jax.ShapeDtypeStruct((E, L), jnp.float32),
        ),
    )(sr, edge, wee, we2, b2, g, beta)


def _edge_mlp_half(sr, edge_h, wee, we2, b2, g, beta, half, oe_prev=None):
    nblk = _EH // _EBLK2
    grid = (nblk,)
    blk = lambda i: (i, 0)
    zero = lambda i: (0, 0)
    oe_map = lambda i: (i + half * nblk, 0)
    in_specs = [
        pl.BlockSpec((_EBLK2, L), blk),
        pl.BlockSpec((_EBLK2, L), blk),
        pl.BlockSpec((L, L), zero),
        pl.BlockSpec((L, L), zero),
        pl.BlockSpec((1, L), zero),
        pl.BlockSpec((1, L), zero),
        pl.BlockSpec((1, L), zero),
    ]
    args = [sr, edge_h, wee, we2, b2, g, beta]
    kwargs = {}
    if oe_prev is not None:
        in_specs.append(pl.BlockSpec(memory_space=pl.ANY))
        args.append(oe_prev)
        kwargs["input_output_aliases"] = {7: 1}

    def body(sr_ref, e_ref, wee_ref, we2_ref, b2_ref, g_ref, bet_ref,
             *rest):
        ne_ref, oe_ref = rest[-2], rest[-1]
        _edge_body(sr_ref, e_ref, wee_ref, we2_ref, b2_ref, g_ref, bet_ref,
                   ne_ref, oe_ref)

    return pl.pallas_call(
        body,
        grid=grid,
        in_specs=in_specs,
        out_specs=[
            pl.BlockSpec((_EBLK2, L), blk),
            pl.BlockSpec((_EBLK2, L), oe_map),
        ],
        out_shape=(
            jax.ShapeDtypeStruct((_EH, L), jnp.float32),
            jax.ShapeDtypeStruct((E, L), jnp.float32),
        ),
        **kwargs,
    )(*args)


def _make_scatter_body(n_edges):
  nchunk = n_edges // _CHUNK
  per_w = -(-nchunk // _NW)
  rem = nchunk - (per_w - 1) * _NW

  def _scatter_body(ne, receivers, p0, p1, ridx, buf, acc, semi, sem):
    cid = lax.axis_index("c")
    sid = lax.axis_index("s")
    wid = sid * _NC + cid

    def zrow(r, carry):
        for q in range(L // 16):
            buf[0][r, pl.ds(q * 16, 16)] = jnp.zeros((16,), jnp.float32)
        return carry

    lax.fori_loop(0, _CHUNK, zrow, 0)
    for kk in range(_ROWS_PER_SUB // _CHUNK):
        pltpu.sync_copy(
            buf[0], acc.at[pl.ds(sid * _ROWS_PER_SUB + kk * _CHUNK, _CHUNK)]
        )
    plsc.subcore_barrier()

    nw = jnp.where(wid < rem, per_w, per_w - 1)

    def issue(j, slot):
        base = (wid + _NW * j) * _CHUNK
        pltpu.async_copy(receivers.at[pl.ds(base, _CHUNK)], ridx[slot], semi[slot])
        pltpu.async_copy(ne.at[pl.ds(base, _CHUNK)], buf[slot], sem[slot])

    def process(j, slot):
        base = (wid + _NW * j) * _CHUNK
        pltpu.make_async_copy(
            receivers.at[pl.ds(base, _CHUNK)], ridx[slot], semi[slot]
        ).wait()
        pltpu.make_async_copy(
            ne.at[pl.ds(base, _CHUNK)], buf[slot], sem[slot]
        ).wait()
        pltpu.sync_copy(buf[slot], acc.at[ridx[slot]], add=True)

    issue(0, 0)

    def pair_body(p, carry):
        for k in range(2):
            j = 2 * p + k
            jn = j + 1

            @pl.when(jn < nw)
            def _():
                issue(jn, 1 - k)

            @pl.when(j < nw)
            def _():
                process(j, k)

        return carry

    lax.fori_loop(0, -(-per_w // 2), pair_body, 0)
    plsc.subcore_barrier()

    for kk in range(_ROWS_PER_SUB // _CHUNK):
        r0 = sid * _ROWS_PER_SUB + kk * _CHUNK

        @pl.when(cid == 0)
        def _():
            pltpu.sync_copy(acc.at[pl.ds(r0, _CHUNK)], p0.at[pl.ds(r0, _CHUNK)])

        @pl.when(cid == 1)
        def _():
            pltpu.sync_copy(acc.at[pl.ds(r0, _CHUNK)], p1.at[pl.ds(r0, _CHUNK)])

  return _scatter_body


def _scatter_add(ne, receivers, n_edges):
    k = functools.partial(
        pl.kernel,
        out_type=(
            jax.ShapeDtypeStruct((_NPAD, L), jnp.float32),
            jax.ShapeDtypeStruct((_NPAD, L), jnp.float32),
        ),
        mesh=_mesh(),
        scratch_types=[
            [pltpu.VMEM((_CHUNK,), jnp.int32)] * 2,
            [pltpu.VMEM((_CHUNK, L), jnp.float32)] * 2,
            pltpu.VMEM_SHARED((_NPAD, L), jnp.float32),
            [pltpu.SemaphoreType.DMA] * 2,
            [pltpu.SemaphoreType.DMA] * 2,
        ],
    )(_make_scatter_body(n_edges))
    return k(ne, receivers)


def _node_body(n_ref, *refs):
    (p_refs, (wna_ref, wnb_ref, b1_ref, w2_ref, b2_ref, g_ref, bet_ref,
              out_ref)) = refs[:-8], refs[-8:]
    x = n_ref[...]
    aggr = p_refs[0][0:N, :] + p_refs[1][0:N, :]
    for p in p_refs[2:]:
        aggr = aggr + p[0:N, :]
    h = (
        jnp.dot(x, wna_ref[...], preferred_element_type=jnp.float32)
        + jnp.dot(aggr, wnb_ref[...], preferred_element_type=jnp.float32)
        + b1_ref[...]
    )
    h = jnp.maximum(h, 0.0)
    h2 = jnp.dot(h, w2_ref[...], preferred_element_type=jnp.float32) + b2_ref[...]
    h2 = jnp.maximum(h2, 0.0)
    m = jnp.mean(h2, axis=-1, keepdims=True)
    cdev = h2 - m
    v = jnp.mean(cdev * cdev, axis=-1, keepdims=True)
    out_ref[...] = cdev * lax.rsqrt(v + 1e-5) * g_ref[...] + bet_ref[...] + x


def _node_mlp(node, partials, wna, wnb, b1, w2, b2, g, beta):
    partials = [p for p in partials if p is not None]
    return pl.pallas_call(
        _node_body,
        out_shape=jax.ShapeDtypeStruct((N, L), jnp.float32),
    )(node, *partials, wna, wnb, b1, w2, b2, g, beta)


def kernel(node_latents, mesh_edge_latents, senders, receivers,
           W_e1, b_e1, W_e2, b_e2, g_e, beta_e,
           W_n1, b_n1, W_n2, b_n2, g_n, beta_n):
    node = node_latents.reshape(N, L)
    edge = mesh_edge_latents.reshape(E, L)
    snd = senders.astype(jnp.int32)
    rcv = receivers.astype(jnp.int32)

    ws = W_e1[0:L, :]
    wr = W_e1[L:2 * L, :]
    wee = W_e1[2 * L:3 * L, :]

    b2e = b_e2.reshape(1, L)
    ge = g_e.reshape(1, L)
    be = beta_e.reshape(1, L)

    sproj, rproj = _proj(node, ws, wr, b_e1.reshape(1, L))
    sr = _gather_sum(sproj, rproj, snd, rcv, E)
    ne, new_edge = _edge_mlp(sr, edge, wee, W_e2, b2e, ge, be)
    p0, p1 = _scatter_add(ne, rcv, E)
    new_node = _node_mlp(node, (p0, p1, None, None), W_n1[0:L, :],
                         W_n1[L:2 * L, :], b_n1.reshape(1, L), W_n2,
                         b_n2.reshape(1, L), g_n.reshape(1, L),
                         beta_n.reshape(1, L))
    return new_node.reshape(1, N, L), new_edge.reshape(1, E, L)


# speedup vs baseline: 1.2194x; 1.0588x over previous
import functools

import jax
import jax.numpy as jnp
from jax import lax
from jax.experimental import pallas as pl
from jax.experimental.pallas import tpu as pltpu
from jax.experimental.pallas import tpu_sc as plsc

N = 10000
E = 320000
L = 128

_NC = 2
_NS = 16
_NW = _NC * _NS

_CHUNK = 128
_NBUF = 3
_NCHUNK = E // _CHUNK
_PER_W = -(-_NCHUNK // _NW)
_REM = _NCHUNK - (_PER_W - 1) * _NW

_NPAD = 10240
_ROWS_PER_SUB = _NPAD // _NS

_EBLK = 4000
_NHALF = 2
_EH = E // _NHALF
_EBLK2 = 2000


def _mesh():
    return plsc.VectorSubcoreMesh(core_axis_name="c", subcore_axis_name="s")


def _proj_body(n_ref, ws_ref, wr_ref, b_ref, s_out, r_out):
    x = n_ref[...]
    s_out[...] = jnp.dot(x, ws_ref[...], preferred_element_type=jnp.float32)
    r_out[...] = (
        jnp.dot(x, wr_ref[...], preferred_element_type=jnp.float32) + b_ref[...]
    )


def _proj(node, ws, wr, b):
    return pl.pallas_call(
        _proj_body,
        out_shape=(
            jax.ShapeDtypeStruct((N, L), jnp.float32),
            jax.ShapeDtypeStruct((N, L), jnp.float32),
        ),
    )(node, ws, wr, b)


def _make_gather_body(n_edges):
  nchunk = n_edges // _CHUNK
  per_w = -(-nchunk // _NW)
  rem = nchunk - (per_w - 1) * _NW

  def _gather_body(sproj, rproj, senders, receivers, out,
                   sidx, ridx, sbuf, rbuf, semi, semg, semw):
    wid = lax.axis_index("s") * _NC + lax.axis_index("c")
    nw = jnp.where(wid < rem, per_w, per_w - 1)

    def idx_load(j, slot):
        base = (wid + _NW * j) * _CHUNK
        pltpu.async_copy(senders.at[pl.ds(base, _CHUNK)], sidx[slot], semi[slot])
        pltpu.async_copy(receivers.at[pl.ds(base, _CHUNK)], ridx[slot], semi[slot])

    def gathers(j, slot):
        base = (wid + _NW * j) * _CHUNK
        pltpu.make_async_copy(
            senders.at[pl.ds(base, _CHUNK)], sidx[slot], semi[slot]).wait()
        pltpu.make_async_copy(
            receivers.at[pl.ds(base, _CHUNK)], ridx[slot], semi[slot]).wait()
        pltpu.async_copy(sproj.at[sidx[slot]], sbuf[slot], semg[slot])
        pltpu.async_copy(rproj.at[ridx[slot]], rbuf[slot], semg[slot])

    def process(j, slot):
        base = (wid + _NW * j) * _CHUNK
        pltpu.make_async_copy(sproj.at[sidx[slot]], sbuf[slot], semg[slot]).wait()
        pltpu.make_async_copy(rproj.at[ridx[slot]], rbuf[slot], semg[slot]).wait()

        def row_body(r, carry2):
            for q in range(L // 16):
                sl = pl.ds(q * 16, 16)
                plsc.addupdate(sbuf[slot].at[r, sl], rbuf[slot][r, sl])
            return carry2

        lax.fori_loop(0, _CHUNK, row_body, 0)
        pltpu.async_copy(sbuf[slot], out.at[pl.ds(base, _CHUNK)], semw[slot])

    idx_load(0, 0)
    idx_load(1, 1)
    idx_load(2, 2)
    gathers(0, 0)
    gathers(1, 1)

    def triple_body(p, carry):
        for k in range(_NBUF):
            j = _NBUF * p + k
            jn = j + 2
            slot_n = (k + 2) % _NBUF

            @pl.when(jn < nw)
            def _():
                @pl.when(jn >= _NBUF)
                def _():
                    pltpu.make_async_copy(
                        sbuf[slot_n], out.at[pl.ds(0, _CHUNK)], semw[slot_n]
                    ).wait()

                gathers(jn, slot_n)

            @pl.when(j < nw)
            def _():
                process(j, k)

            @pl.when(j + _NBUF < nw)
            def _():
                idx_load(j + _NBUF, k)

        return carry

    lax.fori_loop(0, -(-per_w // _NBUF), triple_body, 0)

    for sl in range(_NBUF):
        pltpu.make_async_copy(
            sbuf[sl], out.at[pl.ds(0, _CHUNK)], semw[sl]
        ).wait()

  return _gather_body


def _gather_sum(sproj, rproj, senders, receivers, n_edges):
    k = functools.partial(
        pl.kernel,
        out_type=jax.ShapeDtypeStruct((n_edges, L), jnp.float32),
        mesh=_mesh(),
        scratch_types=[
            [pltpu.VMEM((_CHUNK,), jnp.int32)] * _NBUF,
            [pltpu.VMEM((_CHUNK,), jnp.int32)] * _NBUF,
            [pltpu.VMEM((_CHUNK, L), jnp.float32)] * _NBUF,
            [pltpu.VMEM((_CHUNK, L), jnp.float32)] * _NBUF,
            [pltpu.SemaphoreType.DMA] * _NBUF,
            [pltpu.SemaphoreType.DMA] * _NBUF,
            [pltpu.SemaphoreType.DMA] * _NBUF,
        ],
    )(_make_gather_body(n_edges))
    return k(sproj, rproj, senders, receivers)


def _edge_body(sr_ref, e_ref, wee_ref, we2_ref, b2_ref, g_ref, bet_ref,
               ne_ref, oe_ref):
    e = e_ref[...]
    h = sr_ref[...] + jnp.dot(
        e.astype(jnp.bfloat16), wee_ref[...].astype(jnp.bfloat16),
        preferred_element_type=jnp.float32)
    h = jnp.maximum(h, 0.0)
    h2 = jnp.dot(
        h.astype(jnp.bfloat16), we2_ref[...].astype(jnp.bfloat16),
        preferred_element_type=jnp.float32) + b2_ref[...]
    h2 = jnp.maximum(h2, 0.0)
    m = jnp.mean(h2, axis=-1, keepdims=True)
    cdev = h2 - m
    v = jnp.mean(cdev * cdev, axis=-1, keepdims=True)
    ne = cdev * lax.rsqrt(v + 1e-5) * g_ref[...] + bet_ref[...]
    ne_ref[...] = ne
    oe_ref[...] = ne + e


def _edge_mlp(sr, edge, wee, we2, b2, g, beta):
    grid = (E // _EBLK,)
    blk = lambda i: (i, 0)
    zero = lambda i: (0, 0)
    return pl.pallas_call(
        _edge_body,
        grid=grid,
        in_specs=[
            pl.BlockSpec((_EBLK, L), blk),
            pl.BlockSpec((_EBLK, L), blk),
            pl.BlockSpec((L, L), zero),
            pl.BlockSpec((L, L), zero),
            pl.BlockSpec((1, L), zero),
            pl.BlockSpec((1, L), zero),
            pl.BlockSpec((1, L), zero),
        ],
        out_specs=[
            pl.BlockSpec((_EBLK, L), blk),
            pl.BlockSpec((_EBLK, L), blk),
        ],
        out_shape=(
            jax.ShapeDtypeStruct((E, L), jnp.float32),
            jax.ShapeDtypeStruct((E, L), jnp.float32),
        ),
    )(sr, edge, wee, we2, b2, g, beta)


def _edge_mlp_half(sr, edge_h, wee, we2, b2, g, beta, half, oe_prev=None):
    nblk = _EH // _EBLK2
    grid = (nblk,)
    blk = lambda i: (i, 0)
    zero = lambda i: (0, 0)
    oe_map = lambda i: (i + half * nblk, 0)
    in_specs = [
        pl.BlockSpec((_EBLK2, L), blk),
        pl.BlockSpec((_EBLK2, L), blk),
        pl.BlockSpec((L, L), zero),
        pl.BlockSpec((L, L), zero),
        pl.BlockSpec((1, L), zero),
        pl.BlockSpec((1, L), zero),
        pl.BlockSpec((1, L), zero),
    ]
    args = [sr, edge_h, wee, we2, b2, g, beta]
    kwargs = {}
    if oe_prev is not None:
        in_specs.append(pl.BlockSpec(memory_space=pl.ANY))
        args.append(oe_prev)
        kwargs["input_output_aliases"] = {7: 1}

    def body(sr_ref, e_ref, wee_ref, we2_ref, b2_ref, g_ref, bet_ref,
             *rest):
        ne_ref, oe_ref = rest[-2], rest[-1]
        _edge_body(sr_ref, e_ref, wee_ref, we2_ref, b2_ref, g_ref, bet_ref,
                   ne_ref, oe_ref)

    return pl.pallas_call(
        body,
        grid=grid,
        in_specs=in_specs,
        out_specs=[
            pl.BlockSpec((_EBLK2, L), blk),
            pl.BlockSpec((_EBLK2, L), oe_map),
        ],
        out_shape=(
            jax.ShapeDtypeStruct((_EH, L), jnp.float32),
            jax.ShapeDtypeStruct((E, L), jnp.float32),
        ),
        **kwargs,
    )(*args)


def _make_scatter_body(n_edges):
  nchunk = n_edges // _CHUNK
  per_w = -(-nchunk // _NW)
  rem = nchunk - (per_w - 1) * _NW

  def _scatter_body(ne, receivers, p0, p1, ridx, buf, acc, semi, sem):
    cid = lax.axis_index("c")
    sid = lax.axis_index("s")
    wid = sid * _NC + cid

    def zrow(r, carry):
        for q in range(L // 16):
            buf[0][r, pl.ds(q * 16, 16)] = jnp.zeros((16,), jnp.float32)
        return carry

    lax.fori_loop(0, _CHUNK, zrow, 0)
    for kk in range(_ROWS_PER_SUB // _CHUNK):
        pltpu.sync_copy(
            buf[0], acc.at[pl.ds(sid * _ROWS_PER_SUB + kk * _CHUNK, _CHUNK)]
        )
    plsc.subcore_barrier()

    nw = jnp.where(wid < rem, per_w, per_w - 1)

    def issue(j, slot):
        base = (wid + _NW * j) * _CHUNK
        pltpu.async_copy(receivers.at[pl.ds(base, _CHUNK)], ridx[slot], semi[slot])
        pltpu.async_copy(ne.at[pl.ds(base, _CHUNK)], buf[slot], sem[slot])

    def process(j, slot):
        base = (wid + _NW * j) * _CHUNK
        pltpu.make_async_copy(
            receivers.at[pl.ds(base, _CHUNK)], ridx[slot], semi[slot]
        ).wait()
        pltpu.make_async_copy(
            ne.at[pl.ds(base, _CHUNK)], buf[slot], sem[slot]
        ).wait()
        pltpu.sync_copy(buf[slot], acc.at[ridx[slot]], add=True)

    issue(0, 0)

    def pair_body(p, carry):
        for k in range(2):
            j = 2 * p + k
            jn = j + 1

            @pl.when(jn < nw)
            def _():
                issue(jn, 1 - k)

            @pl.when(j < nw)
            def _():
                process(j, k)

        return carry

    lax.fori_loop(0, -(-per_w // 2), pair_body, 0)
    plsc.subcore_barrier()

    for kk in range(_ROWS_PER_SUB // _CHUNK):
        r0 = sid * _ROWS_PER_SUB + kk * _CHUNK

        @pl.when(cid == 0)
        def _():
            pltpu.sync_copy(acc.at[pl.ds(r0, _CHUNK)], p0.at[pl.ds(r0, _CHUNK)])

        @pl.when(cid == 1)
        def _():
            pltpu.sync_copy(acc.at[pl.ds(r0, _CHUNK)], p1.at[pl.ds(r0, _CHUNK)])

  return _scatter_body


def _scatter_add(ne, receivers, n_edges):
    k = functools.partial(
        pl.kernel,
        out_type=(
            jax.ShapeDtypeStruct((_NPAD, L), jnp.float32),
            jax.ShapeDtypeStruct((_NPAD, L), jnp.float32),
        ),
        mesh=_mesh(),
        scratch_types=[
            [pltpu.VMEM((_CHUNK,), jnp.int32)] * 2,
            [pltpu.VMEM((_CHUNK, L), jnp.float32)] * 2,
            pltpu.VMEM_SHARED((_NPAD, L), jnp.float32),
            [pltpu.SemaphoreType.DMA] * 2,
            [pltpu.SemaphoreType.DMA] * 2,
        ],
    )(_make_scatter_body(n_edges))
    return k(ne, receivers)


def _node_body(n_ref, *refs):
    (p_refs, (wna_ref, wnb_ref, b1_ref, w2_ref, b2_ref, g_ref, bet_ref,
              out_ref)) = refs[:-8], refs[-8:]
    x = n_ref[...]
    aggr = p_refs[0][0:N, :] + p_refs[1][0:N, :]
    for p in p_refs[2:]:
        aggr = aggr + p[0:N, :]
    h = (
        jnp.dot(x, wna_ref[...], preferred_element_type=jnp.float32)
        + jnp.dot(aggr, wnb_ref[...], preferred_element_type=jnp.float32)
        + b1_ref[...]
    )
    h = jnp.maximum(h, 0.0)
    h2 = jnp.dot(h, w2_ref[...], preferred_element_type=jnp.float32) + b2_ref[...]
    h2 = jnp.maximum(h2, 0.0)
    m = jnp.mean(h2, axis=-1, keepdims=True)
    cdev = h2 - m
    v = jnp.mean(cdev * cdev, axis=-1, keepdims=True)
    out_ref[...] = cdev * lax.rsqrt(v + 1e-5) * g_ref[...] + bet_ref[...] + x


def _node_mlp(node, partials, wna, wnb, b1, w2, b2, g, beta):
    partials = [p for p in partials if p is not None]
    return pl.pallas_call(
        _node_body,
        out_shape=jax.ShapeDtypeStruct((N, L), jnp.float32),
    )(node, *partials, wna, wnb, b1, w2, b2, g, beta)


def kernel(node_latents, mesh_edge_latents, senders, receivers,
           W_e1, b_e1, W_e2, b_e2, g_e, beta_e,
           W_n1, b_n1, W_n2, b_n2, g_n, beta_n):
    node = node_latents.reshape(N, L)
    edge = mesh_edge_latents.reshape(E, L)
    snd = senders.astype(jnp.int32)
    rcv = receivers.astype(jnp.int32)

    ws = W_e1[0:L, :]
    wr = W_e1[L:2 * L, :]
    wee = W_e1[2 * L:3 * L, :]

    b2e = b_e2.reshape(1, L)
    ge = g_e.reshape(1, L)
    be = beta_e.reshape(1, L)

    sproj, rproj = _proj(node, ws, wr, b_e1.reshape(1, L))
    sr = _gather_sum(sproj, rproj, snd, rcv, E)
    ne, new_edge = _edge_mlp(sr, edge, wee, W_e2, b2e, ge, be)
    p0, p1 = _scatter_add(ne, rcv, E)
    new_node = _node_mlp(node, (p0, p1, None, None), W_n1[0:L, :],
                         W_n1[L:2 * L, :], b_n1.reshape(1, L), W_n2,
                         b_n2.reshape(1, L), g_n.reshape(1, L),
                         beta_n.reshape(1, L))
    return new_node.reshape(1, N, L), new_edge.reshape(1, E, L)


# speedup vs baseline: 1.2478x; 1.0233x over previous
import functools

import jax
import jax.numpy as jnp
from jax import lax
from jax.experimental import pallas as pl
from jax.experimental.pallas import tpu as pltpu
from jax.experimental.pallas import tpu_sc as plsc

N = 10000
E = 320000
L = 128

_NC = 2
_NS = 16
_NW = _NC * _NS

_CHUNK = 128
_NBUF = 3
_NCHUNK = E // _CHUNK
_PER_W = -(-_NCHUNK // _NW)
_REM = _NCHUNK - (_PER_W - 1) * _NW

_NPAD = 10240
_ROWS_PER_SUB = _NPAD // _NS

_EBLK = 8000
_NHALF = 2
_EH = E // _NHALF
_EBLK2 = 2000


def _mesh():
    return plsc.VectorSubcoreMesh(core_axis_name="c", subcore_axis_name="s")


def _proj_body(n_ref, ws_ref, wr_ref, b_ref, s_out, r_out):
    x = n_ref[...]
    s_out[...] = jnp.dot(x, ws_ref[...], preferred_element_type=jnp.float32)
    r_out[...] = (
        jnp.dot(x, wr_ref[...], preferred_element_type=jnp.float32) + b_ref[...]
    )


def _proj(node, ws, wr, b):
    return pl.pallas_call(
        _proj_body,
        out_shape=(
            jax.ShapeDtypeStruct((N, L), jnp.float32),
            jax.ShapeDtypeStruct((N, L), jnp.float32),
        ),
    )(node, ws, wr, b)


def _make_gather_body(n_edges):
  nchunk = n_edges // _CHUNK
  per_w = -(-nchunk // _NW)
  rem = nchunk - (per_w - 1) * _NW

  def _gather_body(sproj, rproj, senders, receivers, out,
                   sidx, ridx, sbuf, rbuf, semi, semg, semw):
    wid = lax.axis_index("s") * _NC + lax.axis_index("c")
    nw = jnp.where(wid < rem, per_w, per_w - 1)

    def idx_load(j, slot):
        base = (wid + _NW * j) * _CHUNK
        pltpu.async_copy(senders.at[pl.ds(base, _CHUNK)], sidx[slot], semi[slot])
        pltpu.async_copy(receivers.at[pl.ds(base, _CHUNK)], ridx[slot], semi[slot])

    def gathers(j, slot):
        base = (wid + _NW * j) * _CHUNK
        pltpu.make_async_copy(
            senders.at[pl.ds(base, _CHUNK)], sidx[slot], semi[slot]).wait()
        pltpu.make_async_copy(
            receivers.at[pl.ds(base, _CHUNK)], ridx[slot], semi[slot]).wait()
        pltpu.async_copy(sproj.at[sidx[slot]], sbuf[slot], semg[slot])
        pltpu.async_copy(rproj.at[ridx[slot]], rbuf[slot], semg[slot])

    def process(j, slot):
        base = (wid + _NW * j) * _CHUNK
        pltpu.make_async_copy(sproj.at[sidx[slot]], sbuf[slot], semg[slot]).wait()
        pltpu.make_async_copy(rproj.at[ridx[slot]], rbuf[slot], semg[slot]).wait()

        def row_body(r, carry2):
            for q in range(L // 16):
                sl = pl.ds(q * 16, 16)
                plsc.addupdate(sbuf[slot].at[r, sl], rbuf[slot][r, sl])
            return carry2

        lax.fori_loop(0, _CHUNK, row_body, 0)
        pltpu.async_copy(sbuf[slot], out.at[pl.ds(base, _CHUNK)], semw[slot])

    idx_load(0, 0)
    idx_load(1, 1)
    idx_load(2, 2)
    gathers(0, 0)
    gathers(1, 1)

    def triple_body(p, carry):
        for k in range(_NBUF):
            j = _NBUF * p + k
            jn = j + 2
            slot_n = (k + 2) % _NBUF

            @pl.when(jn < nw)
            def _():
                @pl.when(jn >= _NBUF)
                def _():
                    pltpu.make_async_copy(
                        sbuf[slot_n], out.at[pl.ds(0, _CHUNK)], semw[slot_n]
                    ).wait()

                gathers(jn, slot_n)

            @pl.when(j < nw)
            def _():
                process(j, k)

            @pl.when(j + _NBUF < nw)
            def _():
                idx_load(j + _NBUF, k)

        return carry

    lax.fori_loop(0, -(-per_w // _NBUF), triple_body, 0)

    for sl in range(_NBUF):
        pltpu.make_async_copy(
            sbuf[sl], out.at[pl.ds(0, _CHUNK)], semw[sl]
        ).wait()

  return _gather_body


def _gather_sum(sproj, rproj, senders, receivers, n_edges):
    k = functools.partial(
        pl.kernel,
        out_type=jax.ShapeDtypeStruct((n_edges, L), jnp.float32),
        mesh=_mesh(),
        scratch_types=[
            [pltpu.VMEM((_CHUNK,), jnp.int32)] * _NBUF,
            [pltpu.VMEM((_CHUNK,), jnp.int32)] * _NBUF,
            [pltpu.VMEM((_CHUNK, L), jnp.float32)] * _NBUF,
            [pltpu.VMEM((_CHUNK, L), jnp.float32)] * _NBUF,
            [pltpu.SemaphoreType.DMA] * _NBUF,
            [pltpu.SemaphoreType.DMA] * _NBUF,
            [pltpu.SemaphoreType.DMA] * _NBUF,
        ],
    )(_make_gather_body(n_edges))
    return k(sproj, rproj, senders, receivers)


def _edge_body(sr_ref, e_ref, wee_ref, we2_ref, b2_ref, g_ref, bet_ref,
               ne_ref, oe_ref):
    e = e_ref[...]
    h = sr_ref[...] + jnp.dot(
        e.astype(jnp.bfloat16), wee_ref[...].astype(jnp.bfloat16),
        preferred_element_type=jnp.float32)
    h = jnp.maximum(h, 0.0)
    h2 = jnp.dot(
        h.astype(jnp.bfloat16), we2_ref[...].astype(jnp.bfloat16),
        preferred_element_type=jnp.float32) + b2_ref[...]
    h2 = jnp.maximum(h2, 0.0)
    m = jnp.mean(h2, axis=-1, keepdims=True)
    cdev = h2 - m
    v = jnp.mean(cdev * cdev, axis=-1, keepdims=True)
    ne = cdev * lax.rsqrt(v + 1e-5) * g_ref[...] + bet_ref[...]
    ne_ref[...] = ne
    oe_ref[...] = ne + e


def _edge_mlp(sr, edge, wee, we2, b2, g, beta):
    grid = (E // _EBLK,)
    blk = lambda i: (i, 0)
    zero = lambda i: (0, 0)
    return pl.pallas_call(
        _edge_body,
        grid=grid,
        in_specs=[
            pl.BlockSpec((_EBLK, L), blk),
            pl.BlockSpec((_EBLK, L), blk),
            pl.BlockSpec((L, L), zero),
            pl.BlockSpec((L, L), zero),
            pl.BlockSpec((1, L), zero),
            pl.BlockSpec((1, L), zero),
            pl.BlockSpec((1, L), zero),
        ],
        out_specs=[
            pl.BlockSpec((_EBLK, L), blk),
            pl.BlockSpec((_EBLK, L), blk),
        ],
        out_shape=(
            jax.ShapeDtypeStruct((E, L), jnp.float32),
            jax.ShapeDtypeStruct((E, L), jnp.float32),
        ),
    )(sr, edge, wee, we2, b2, g, beta)


def _edge_mlp_half(sr, edge_h, wee, we2, b2, g, beta, half, oe_prev=None):
    nblk = _EH // _EBLK2
    grid = (nblk,)
    blk = lambda i: (i, 0)
    zero = lambda i: (0, 0)
    oe_map = lambda i: (i + half * nblk, 0)
    in_specs = [
        pl.BlockSpec((_EBLK2, L), blk),
        pl.BlockSpec((_EBLK2, L), blk),
        pl.BlockSpec((L, L), zero),
        pl.BlockSpec((L, L), zero),
        pl.BlockSpec((1, L), zero),
        pl.BlockSpec((1, L), zero),
        pl.BlockSpec((1, L), zero),
    ]
    args = [sr, edge_h, wee, we2, b2, g, beta]
    kwargs = {}
    if oe_prev is not None:
        in_specs.append(pl.BlockSpec(memory_space=pl.ANY))
        args.append(oe_prev)
        kwargs["input_output_aliases"] = {7: 1}

    def body(sr_ref, e_ref, wee_ref, we2_ref, b2_ref, g_ref, bet_ref,
             *rest):
        ne_ref, oe_ref = rest[-2], rest[-1]
        _edge_body(sr_ref, e_ref, wee_ref, we2_ref, b2_ref, g_ref, bet_ref,
                   ne_ref, oe_ref)

    return pl.pallas_call(
        body,
        grid=grid,
        in_specs=in_specs,
        out_specs=[
            pl.BlockSpec((_EBLK2, L), blk),
            pl.BlockSpec((_EBLK2, L), oe_map),
        ],
        out_shape=(
            jax.ShapeDtypeStruct((_EH, L), jnp.float32),
            jax.ShapeDtypeStruct((E, L), jnp.float32),
        ),
        **kwargs,
    )(*args)


def _make_scatter_body(n_edges):
  nchunk = n_edges // _CHUNK
  per_w = -(-nchunk // _NW)
  rem = nchunk - (per_w - 1) * _NW

  def _scatter_body(ne, receivers, p0, p1, ridx, buf, acc, semi, sem):
    cid = lax.axis_index("c")
    sid = lax.axis_index("s")
    wid = sid * _NC + cid

    def zrow(r, carry):
        for q in range(L // 16):
            buf[0][r, pl.ds(q * 16, 16)] = jnp.zeros((16,), jnp.float32)
        return carry

    lax.fori_loop(0, _CHUNK, zrow, 0)
    for kk in range(_ROWS_PER_SUB // _CHUNK):
        pltpu.sync_copy(
            buf[0], acc.at[pl.ds(sid * _ROWS_PER_SUB + kk * _CHUNK, _CHUNK)]
        )
    plsc.subcore_barrier()

    nw = jnp.where(wid < rem, per_w, per_w - 1)

    def issue(j, slot):
        base = (wid + _NW * j) * _CHUNK
        pltpu.async_copy(receivers.at[pl.ds(base, _CHUNK)], ridx[slot], semi[slot])
        pltpu.async_copy(ne.at[pl.ds(base, _CHUNK)], buf[slot], sem[slot])

    def process(j, slot):
        base = (wid + _NW * j) * _CHUNK
        pltpu.make_async_copy(
            receivers.at[pl.ds(base, _CHUNK)], ridx[slot], semi[slot]
        ).wait()
        pltpu.make_async_copy(
            ne.at[pl.ds(base, _CHUNK)], buf[slot], sem[slot]
        ).wait()
        pltpu.sync_copy(buf[slot], acc.at[ridx[slot]], add=True)

    issue(0, 0)

    def pair_body(p, carry):
        for k in range(2):
            j = 2 * p + k
            jn = j + 1

            @pl.when(jn < nw)
            def _():
                issue(jn, 1 - k)

            @pl.when(j < nw)
            def _():
                process(j, k)

        return carry

    lax.fori_loop(0, -(-per_w // 2), pair_body, 0)
    plsc.subcore_barrier()

    for kk in range(_ROWS_PER_SUB // _CHUNK):
        r0 = sid * _ROWS_PER_SUB + kk * _CHUNK

        @pl.when(cid == 0)
        def _():
            pltpu.sync_copy(acc.at[pl.ds(r0, _CHUNK)], p0.at[pl.ds(r0, _CHUNK)])

        @pl.when(cid == 1)
        def _():
            pltpu.sync_copy(acc.at[pl.ds(r0, _CHUNK)], p1.at[pl.ds(r0, _CHUNK)])

  return _scatter_body


def _scatter_add(ne, receivers, n_edges):
    k = functools.partial(
        pl.kernel,
        out_type=(
            jax.ShapeDtypeStruct((_NPAD, L), jnp.float32),
            jax.ShapeDtypeStruct((_NPAD, L), jnp.float32),
        ),
        mesh=_mesh(),
        scratch_types=[
            [pltpu.VMEM((_CHUNK,), jnp.int32)] * 2,
            [pltpu.VMEM((_CHUNK, L), jnp.float32)] * 2,
            pltpu.VMEM_SHARED((_NPAD, L), jnp.float32),
            [pltpu.SemaphoreType.DMA] * 2,
            [pltpu.SemaphoreType.DMA] * 2,
        ],
    )(_make_scatter_body(n_edges))
    return k(ne, receivers)


def _node_body(n_ref, *refs):
    (p_refs, (wna_ref, wnb_ref, b1_ref, w2_ref, b2_ref, g_ref, bet_ref,
              out_ref)) = refs[:-8], refs[-8:]
    x = n_ref[...]
    aggr = p_refs[0][0:N, :] + p_refs[1][0:N, :]
    for p in p_refs[2:]:
        aggr = aggr + p[0:N, :]
    h = (
        jnp.dot(x, wna_ref[...], preferred_element_type=jnp.float32)
        + jnp.dot(aggr, wnb_ref[...], preferred_element_type=jnp.float32)
        + b1_ref[...]
    )
    h = jnp.maximum(h, 0.0)
    h2 = jnp.dot(h, w2_ref[...], preferred_element_type=jnp.float32) + b2_ref[...]
    h2 = jnp.maximum(h2, 0.0)
    m = jnp.mean(h2, axis=-1, keepdims=True)
    cdev = h2 - m
    v = jnp.mean(cdev * cdev, axis=-1, keepdims=True)
    out_ref[...] = cdev * lax.rsqrt(v + 1e-5) * g_ref[...] + bet_ref[...] + x


def _node_mlp(node, partials, wna, wnb, b1, w2, b2, g, beta):
    partials = [p for p in partials if p is not None]
    return pl.pallas_call(
        _node_body,
        out_shape=jax.ShapeDtypeStruct((N, L), jnp.float32),
    )(node, *partials, wna, wnb, b1, w2, b2, g, beta)


def kernel(node_latents, mesh_edge_latents, senders, receivers,
           W_e1, b_e1, W_e2, b_e2, g_e, beta_e,
           W_n1, b_n1, W_n2, b_n2, g_n, beta_n):
    node = node_latents.reshape(N, L)
    edge = mesh_edge_latents.reshape(E, L)
    snd = senders.astype(jnp.int32)
    rcv = receivers.astype(jnp.int32)

    ws = W_e1[0:L, :]
    wr = W_e1[L:2 * L, :]
    wee = W_e1[2 * L:3 * L, :]

    b2e = b_e2.reshape(1, L)
    ge = g_e.reshape(1, L)
    be = beta_e.reshape(1, L)

    sproj, rproj = _proj(node, ws, wr, b_e1.reshape(1, L))
    sr = _gather_sum(sproj, rproj, snd, rcv, E)
    ne, new_edge = _edge_mlp(sr, edge, wee, W_e2, b2e, ge, be)
    p0, p1 = _scatter_add(ne, rcv, E)
    new_node = _node_mlp(node, (p0, p1, None, None), W_n1[0:L, :],
                         W_n1[L:2 * L, :], b_n1.reshape(1, L), W_n2,
                         b_n2.reshape(1, L), g_n.reshape(1, L),
                         beta_n.reshape(1, L))
    return new_node.reshape(1, N, L), new_edge.reshape(1, E, L)
